# Initial kernel scaffold; baseline (speedup 1.0000x reference)
import jax
import jax.numpy as jnp
from jax.experimental import pallas as pl


def kernel(x, edge_index, W, b):
    raise NotImplementedError("write your pallas kernel here")


# speedup vs baseline: 20.5777x; 20.5777x over previous
import functools

import jax
import jax.numpy as jnp
from jax import lax
from jax.experimental import pallas as pl
from jax.experimental.pallas import tpu as pltpu
from jax.experimental.pallas import tpu_sc as plsc

N = 10000
D = 128
E = 320000

NT = 32
CH = 128
NCHUNK = 79
EPT = CH * NCHUNK
E_PAD = EPT * NT

N_ACC = 10240
ZROWS = N_ACC // 16

NP_COL = 10240
CPT = NP_COL // 16

_mesh = plsc.VectorSubcoreMesh(core_axis_name="c", subcore_axis_name="s")


def _deg_body(dst_hbm, out_hbm, idx_v, hist_v, chunk_v, red_v, shared):
    c = lax.axis_index("c")
    s = lax.axis_index("s")
    w = c * 16 + s
    pltpu.sync_copy(dst_hbm.at[pl.ds(w * EPT, EPT)], idx_v)

    def zbody(i, _):
        hist_v[pl.ds(i * 16, 16)] = jnp.zeros((16,), jnp.float32)
        return _

    lax.fori_loop(0, NP_COL // 16, zbody, None)

    ones16 = jnp.ones((16,), jnp.float32)

    def hbody(i, _):
        idx16 = idx_v[pl.ds(i * 16, 16)]
        plsc.addupdate_scatter(hist_v, [idx16], ones16)
        return _

    lax.fori_loop(0, EPT // 16, hbody, None)

    pltpu.sync_copy(hist_v, shared.at[s])
    plsc.subcore_barrier()
    pltpu.sync_copy(shared.at[:, pl.ds(s * CPT, CPT)], chunk_v)

    def rbody(v, _):
        t = chunk_v[0, pl.ds(v * 16, 16)]
        for r in range(1, 16):
            t = t + chunk_v[r, pl.ds(v * 16, 16)]
        red_v[pl.ds(v * 16, 16)] = t
        return _

    lax.fori_loop(0, CPT // 16, rbody, None)
    pltpu.sync_copy(red_v, out_hbm.at[c, pl.ds(s * CPT, CPT)])


_deg_call = pl.kernel(
    _deg_body,
    out_type=jax.ShapeDtypeStruct((2, NP_COL), jnp.float32),
    mesh=_mesh,
    compiler_params=pltpu.CompilerParams(needs_layout_passes=False),
    scratch_types=[
        pltpu.VMEM((EPT,), jnp.int32),
        pltpu.VMEM((NP_COL,), jnp.float32),
        pltpu.VMEM((16, CPT), jnp.float32),
        pltpu.VMEM((CPT,), jnp.float32),
        pltpu.VMEM_SHARED((16, NP_COL), jnp.float32),
    ],
)


def _scatter_body(src_hbm, dst3_hbm, g_hbm, z_hbm, out_hbm,
                  srcv, dstv, buf, acc):
    c = lax.axis_index("c")
    s = lax.axis_index("s")
    w = c * 16 + s
    pltpu.sync_copy(src_hbm.at[pl.ds(w * EPT, EPT)], srcv)
    pltpu.sync_copy(dst3_hbm.at[w], dstv)
    pltpu.sync_copy(z_hbm.at[pl.ds(s * ZROWS, ZROWS)],
                    acc.at[pl.ds(s * ZROWS, ZROWS)])
    plsc.subcore_barrier()

    def body(ci, _):
        pltpu.sync_copy(g_hbm.at[srcv.at[pl.ds(ci * CH, CH)]], buf)
        pltpu.sync_copy(buf, acc.at[dstv.at[ci]], add=True)
        return _

    lax.fori_loop(0, NCHUNK, body, None)
    plsc.subcore_barrier()
    pltpu.sync_copy(acc.at[pl.ds(s * ZROWS, ZROWS)],
                    out_hbm.at[c, pl.ds(s * ZROWS, ZROWS)])


_scatter_call = pl.kernel(
    _scatter_body,
    out_type=jax.ShapeDtypeStruct((2, N_ACC, D), jnp.float32),
    mesh=_mesh,
    compiler_params=pltpu.CompilerParams(needs_layout_passes=False),
    scratch_types=[
        pltpu.VMEM((EPT,), jnp.int32),
        pltpu.VMEM((NCHUNK, CH), jnp.int32),
        pltpu.VMEM((CH, D), jnp.float32),
        pltpu.VMEM_SHARED((N_ACC, D), jnp.float32),
    ],
)


def _tca_body(x_ref, w_ref, deg_ref, g_ref):
    dg = deg_ref[0] + deg_ref[1] + 1.0
    dinv = lax.rsqrt(dg)
    h = jnp.dot(x_ref[...], w_ref[...], preferred_element_type=jnp.float32)
    g_ref[...] = h * dinv


def _tcb_body(acc_ref, g_ref, deg_ref, b_ref, o_ref):
    dg = deg_ref[0] + deg_ref[1] + 1.0
    dinv = lax.rsqrt(dg)
    ssum = acc_ref[0] + acc_ref[1] + g_ref[...]
    o_ref[...] = jax.nn.gelu(dinv * ssum + b_ref[...])


_RB = 1000


def _tc_a(x, W, deg2):
    return pl.pallas_call(
        _tca_body,
        grid=(N // _RB,),
        in_specs=[
            pl.BlockSpec((_RB, D), lambda i: (i, 0)),
            pl.BlockSpec((D, D), lambda i: (0, 0)),
            pl.BlockSpec((2, _RB, 1), lambda i: (0, i, 0)),
        ],
        out_specs=pl.BlockSpec((_RB, D), lambda i: (i, 0)),
        out_shape=jax.ShapeDtypeStruct((N, D), jnp.float32),
    )(x, W, deg2)


def _tc_b(accs, g, deg2, b2):
    return pl.pallas_call(
        _tcb_body,
        grid=(N // _RB,),
        in_specs=[
            pl.BlockSpec((2, _RB, D), lambda i: (0, i, 0)),
            pl.BlockSpec((_RB, D), lambda i: (i, 0)),
            pl.BlockSpec((2, _RB, 1), lambda i: (0, i, 0)),
            pl.BlockSpec((1, D), lambda i: (0, 0)),
        ],
        out_specs=pl.BlockSpec((_RB, D), lambda i: (i, 0)),
        out_shape=jax.ShapeDtypeStruct((N, D), jnp.float32),
    )(accs, g, deg2, b2)


@jax.jit
def kernel(x, edge_index, W, b):
    src = edge_index[0]
    dst = edge_index[1]
    pad = E_PAD - E
    src_pad = jnp.concatenate([src, jnp.zeros((pad,), jnp.int32)])
    dst_pad = jnp.concatenate([dst, jnp.full((pad,), N, jnp.int32)])
    dst3 = dst_pad.reshape(NT, NCHUNK, CH)

    deg_out = _deg_call(dst_pad)
    deg2 = deg_out[:, :N].reshape(2, N, 1)

    g = _tc_a(x, W, deg2)

    z = jnp.zeros((N_ACC, D), jnp.float32)
    accs = _scatter_call(src_pad, dst3, g, z)

    return _tc_b(accs, g, deg2, b.reshape(1, D))
